# baseline (speedup 1.0000x reference)
import functools

import jax
import jax.numpy as jnp
from jax import lax
from jax.experimental import pallas as pl
from jax.experimental.pallas import tpu as pltpu
from jax.experimental.pallas import tpu_sc as plsc

_LANE = 128
_G = 8


@functools.cache
def _build(n_workers, chunks, emb_dim):
    steps = chunks // _G
    rows_per_step = _G * _LANE
    rows_per_worker = chunks * _LANE
    total_rows = n_workers * rows_per_worker
    mesh = plsc.VectorSubcoreMesh(core_axis_name="c", subcore_axis_name="s")
    num_cores = plsc.get_sparse_core_info().num_cores

    @functools.partial(
        pl.kernel,
        mesh=mesh,
        out_type=jax.ShapeDtypeStruct((total_rows, emb_dim), jnp.float32),
        scratch_types=[
            pltpu.VMEM((chunks, _LANE), jnp.int32),
            pltpu.VMEM((rows_per_step, emb_dim), jnp.float32),
            pltpu.SemaphoreType.DMA,
        ],
        compiler_params=pltpu.CompilerParams(use_tc_tiling_on_sc=False),
    )
    def gather_kernel(idx_hbm, table_hbm, out_hbm, idx_v, rows_v, sem):
        wid = lax.axis_index("s") * num_cores + lax.axis_index("c")
        pltpu.sync_copy(idx_hbm.at[wid], idx_v)
        out_base = wid * rows_per_worker

        def step(s, carry):
            copies = [
                pltpu.async_copy(
                    table_hbm.at[idx_v.at[s * _G + g]],
                    rows_v.at[pl.ds(g * _LANE, _LANE)],
                    sem,
                )
                for g in range(_G)
            ]
            for c in copies:
                c.wait()
            pltpu.sync_copy(
                rows_v, out_hbm.at[pl.ds(out_base + s * rows_per_step, rows_per_step)]
            )
            return carry

        lax.fori_loop(0, steps, step, 0)

    return gather_kernel


def kernel(input_variable, embedding_weight):
    batch, hist = input_variable.shape
    emb_dim = embedding_weight.shape[1]
    total = batch * hist
    info = plsc.get_sparse_core_info()
    n_workers = info.num_cores * info.num_subcores
    chunks = total // (n_workers * _LANE)
    idx = input_variable.reshape(n_workers, chunks, _LANE)
    out = _build(n_workers, chunks, emb_dim)(idx, embedding_weight)
    return out.reshape(batch, hist, emb_dim)


# speedup vs baseline: 1.0133x; 1.0133x over previous
import functools

import jax
import jax.numpy as jnp
from jax import lax
from jax.experimental import pallas as pl
from jax.experimental.pallas import tpu as pltpu
from jax.experimental.pallas import tpu_sc as plsc

_LANE = 128
_G = 4


@functools.cache
def _build(n_workers, chunks, emb_dim):
    steps = chunks // _G
    rows_per_step = _G * _LANE
    rows_per_worker = chunks * _LANE
    total_rows = n_workers * rows_per_worker
    mesh = plsc.VectorSubcoreMesh(core_axis_name="c", subcore_axis_name="s")
    num_cores = plsc.get_sparse_core_info().num_cores

    @functools.partial(
        pl.kernel,
        mesh=mesh,
        out_type=jax.ShapeDtypeStruct((total_rows, emb_dim), jnp.float32),
        scratch_types=[
            pltpu.VMEM((chunks, _LANE), jnp.int32),
            pltpu.VMEM((rows_per_step, emb_dim), jnp.float32),
            pltpu.VMEM((rows_per_step, emb_dim), jnp.float32),
            pltpu.SemaphoreType.DMA,
            pltpu.SemaphoreType.DMA,
        ],
        compiler_params=pltpu.CompilerParams(use_tc_tiling_on_sc=False),
    )
    def gather_kernel(idx_hbm, table_hbm, out_hbm, idx_v, buf0, buf1, sem0, sem1):
        wid = lax.axis_index("s") * num_cores + lax.axis_index("c")
        pltpu.sync_copy(idx_hbm.at[wid], idx_v)
        out_base = wid * rows_per_worker
        bufs = (buf0, buf1)
        sems = (sem0, sem1)

        def fire(t, buf, sem):
            for g in range(_G):
                pltpu.async_copy(
                    table_hbm.at[idx_v.at[t * _G + g]],
                    buf.at[pl.ds(g * _LANE, _LANE)],
                    sem,
                )

        def drain(buf, sem):
            pltpu.make_async_copy(out_hbm.at[pl.ds(0, rows_per_step)], buf, sem).wait()

        fire(0, buf0, sem0)
        fire(1, buf1, sem1)

        def step2(i, carry):
            tt = i * 2
            for b in range(2):
                t = tt + b
                drain(bufs[b], sems[b])
                pltpu.sync_copy(
                    bufs[b],
                    out_hbm.at[pl.ds(out_base + t * rows_per_step, rows_per_step)],
                )
                fire(t + 2, bufs[b], sems[b])
            return carry

        lax.fori_loop(0, (steps - 2) // 2, step2, 0)

        for b in range(2):
            t = steps - 2 + b
            drain(bufs[b], sems[b])
            pltpu.sync_copy(
                bufs[b],
                out_hbm.at[pl.ds(out_base + t * rows_per_step, rows_per_step)],
            )

    return gather_kernel


def kernel(input_variable, embedding_weight):
    batch, hist = input_variable.shape
    emb_dim = embedding_weight.shape[1]
    total = batch * hist
    info = plsc.get_sparse_core_info()
    n_workers = info.num_cores * info.num_subcores
    chunks = total // (n_workers * _LANE)
    idx = input_variable.reshape(n_workers, chunks, _LANE)
    out = _build(n_workers, chunks, emb_dim)(idx, embedding_weight)
    return out.reshape(batch, hist, emb_dim)


# speedup vs baseline: 1.0910x; 1.0767x over previous
import functools

import jax
import jax.numpy as jnp
from jax import lax
from jax.experimental import pallas as pl
from jax.experimental.pallas import tpu as pltpu
from jax.experimental.pallas import tpu_sc as plsc

_CHUNK = 128
_G = 2
_PAD_BLK = 8000


def _pad_block(src_ref, dst_ref):
    dst_ref[:, : src_ref.shape[1]] = src_ref[...]
    dst_ref[:, src_ref.shape[1] :] = jnp.zeros(
        (src_ref.shape[0], dst_ref.shape[1] - src_ref.shape[1]), src_ref.dtype
    )


@functools.cache
def _build_pad(vocab, emb_dim):
    return pl.pallas_call(
        _pad_block,
        grid=(vocab // _PAD_BLK,),
        in_specs=[pl.BlockSpec((_PAD_BLK, emb_dim), lambda i: (i, 0))],
        out_specs=pl.BlockSpec((_PAD_BLK, 2 * emb_dim), lambda i: (i, 0)),
        out_shape=jax.ShapeDtypeStruct((vocab, 2 * emb_dim), jnp.float32),
    )


@functools.cache
def _build_gather(total, emb_dim, n_workers, num_cores):
    per_worker = total // n_workers
    chunks = per_worker // _CHUNK
    steps = chunks // _G
    rows_per_step = _G * _CHUNK
    mesh = plsc.VectorSubcoreMesh(core_axis_name="c", subcore_axis_name="s")

    @functools.partial(
        pl.kernel,
        mesh=mesh,
        out_type=jax.ShapeDtypeStruct((total, 2 * emb_dim), jnp.float32),
        scratch_types=[
            pltpu.VMEM((chunks, _CHUNK), jnp.int32),
            pltpu.VMEM((rows_per_step, 2 * emb_dim), jnp.float32),
            pltpu.VMEM((rows_per_step, 2 * emb_dim), jnp.float32),
            pltpu.SemaphoreType.DMA,
            pltpu.SemaphoreType.DMA,
        ],
        compiler_params=pltpu.CompilerParams(
            use_tc_tiling_on_sc=True, needs_layout_passes=False
        ),
    )
    def gather_kernel(
        idx_hbm, table_hbm, out_hbm,
        idx_v, buf0, buf1, sem0, sem1,
    ):
        wid = lax.axis_index("s") * num_cores + lax.axis_index("c")
        pltpu.sync_copy(idx_hbm.at[wid], idx_v)
        out_base = wid * per_worker
        bufs = (buf0, buf1)
        sems = (sem0, sem1)

        def fire(t, buf, sem):
            for g in range(_G):
                pltpu.async_copy(
                    table_hbm.at[idx_v.at[t * _G + g]],
                    buf.at[pl.ds(g * _CHUNK, _CHUNK)],
                    sem,
                )

        def drain(buf, sem):
            pltpu.make_async_copy(
                table_hbm.at[pl.ds(0, rows_per_step)], buf, sem
            ).wait()

        def writeback(t, buf):
            pltpu.sync_copy(
                buf,
                out_hbm.at[pl.ds(out_base + t * rows_per_step, rows_per_step)],
            )

        fire(0, buf0, sem0)
        fire(1, buf1, sem1)

        def step2(i, carry):
            tt = i * 2
            for b in range(2):
                t = tt + b
                drain(bufs[b], sems[b])
                writeback(t, bufs[b])
                fire(t + 2, bufs[b], sems[b])
            return carry

        lax.fori_loop(0, (steps - 2) // 2, step2, 0)

        for b in range(2):
            t = steps - 2 + b
            drain(bufs[b], sems[b])
            writeback(t, bufs[b])

    return gather_kernel


def kernel(input_variable, embedding_weight):
    batch, hist = input_variable.shape
    vocab, emb_dim = embedding_weight.shape
    total = batch * hist
    info = plsc.get_sparse_core_info()
    n_workers = info.num_cores * info.num_subcores
    chunks = total // (n_workers * _CHUNK)

    table128 = _build_pad(vocab, emb_dim)(embedding_weight)

    idx = jnp.maximum(input_variable, 0).reshape(n_workers, chunks, _CHUNK)
    out = _build_gather(total, emb_dim, n_workers, info.num_cores)(idx, table128)
    return out[:, :emb_dim].reshape(batch, hist, emb_dim)


# speedup vs baseline: 1.3327x; 1.2216x over previous
import functools

import jax
import jax.numpy as jnp
from jax import lax
from jax.experimental import pallas as pl
from jax.experimental.pallas import tpu as pltpu
from jax.experimental.pallas import tpu_sc as plsc

_CHUNK = 128
_G = 2
_PAD_BLK = 8000


def _pad_block(src_ref, dst_ref):
    dst_ref[:, : src_ref.shape[1]] = src_ref[...]
    dst_ref[:, src_ref.shape[1] :] = jnp.zeros(
        (src_ref.shape[0], dst_ref.shape[1] - src_ref.shape[1]), src_ref.dtype
    )


@functools.cache
def _build_pad(vocab, emb_dim):
    return pl.pallas_call(
        _pad_block,
        grid=(vocab // _PAD_BLK,),
        in_specs=[pl.BlockSpec((_PAD_BLK, emb_dim), lambda i: (i, 0))],
        out_specs=pl.BlockSpec((_PAD_BLK, 2 * emb_dim), lambda i: (i, 0)),
        out_shape=jax.ShapeDtypeStruct((vocab, 2 * emb_dim), jnp.float32),
    )


@functools.cache
def _build_gather(total, emb_dim, n_workers, num_cores):
    per_worker = total // n_workers
    chunks = per_worker // _CHUNK
    steps = chunks // _G
    rows_per_step = _G * _CHUNK
    mesh = plsc.VectorSubcoreMesh(core_axis_name="c", subcore_axis_name="s")

    @functools.partial(
        pl.kernel,
        mesh=mesh,
        out_type=jax.ShapeDtypeStruct((total, emb_dim), jnp.float32),
        scratch_types=[
            pltpu.VMEM((chunks, _CHUNK), jnp.int32),
            pltpu.VMEM((rows_per_step, 2 * emb_dim), jnp.float32),
            pltpu.VMEM((rows_per_step, 2 * emb_dim), jnp.float32),
            pltpu.VMEM((rows_per_step, emb_dim), jnp.float32),
            pltpu.SemaphoreType.DMA,
            pltpu.SemaphoreType.DMA,
        ],
        compiler_params=pltpu.CompilerParams(
            use_tc_tiling_on_sc=True, needs_layout_passes=False
        ),
    )
    def gather_kernel(
        idx_hbm, table_hbm, out_hbm,
        idx_v, buf0, buf1, packed, sem0, sem1,
    ):
        wid = lax.axis_index("s") * num_cores + lax.axis_index("c")
        pltpu.sync_copy(idx_hbm.at[wid], idx_v)
        out_base = wid * per_worker
        bufs = (buf0, buf1)
        sems = (sem0, sem1)

        def fire(t, buf, sem):
            for g in range(_G):
                pltpu.async_copy(
                    table_hbm.at[idx_v.at[t * _G + g]],
                    buf.at[pl.ds(g * _CHUNK, _CHUNK)],
                    sem,
                )

        def drain(buf, sem):
            pltpu.make_async_copy(
                table_hbm.at[pl.ds(0, rows_per_step)], buf, sem
            ).wait()

        def writeback(t, buf):
            def repack(r, carry):
                for g in range(emb_dim // 16):
                    packed[r, pl.ds(g * 16, 16)] = buf[r, pl.ds(g * 16, 16)]
                return carry

            lax.fori_loop(0, rows_per_step, repack, 0)
            pltpu.sync_copy(
                packed,
                out_hbm.at[pl.ds(out_base + t * rows_per_step, rows_per_step)],
            )

        fire(0, buf0, sem0)
        fire(1, buf1, sem1)

        def step2(i, carry):
            tt = i * 2
            for b in range(2):
                t = tt + b
                drain(bufs[b], sems[b])
                writeback(t, bufs[b])
                fire(t + 2, bufs[b], sems[b])
            return carry

        lax.fori_loop(0, (steps - 2) // 2, step2, 0)

        for b in range(2):
            t = steps - 2 + b
            drain(bufs[b], sems[b])
            writeback(t, bufs[b])

    return gather_kernel


def kernel(input_variable, embedding_weight):
    batch, hist = input_variable.shape
    vocab, emb_dim = embedding_weight.shape
    total = batch * hist
    info = plsc.get_sparse_core_info()
    n_workers = info.num_cores * info.num_subcores
    chunks = total // (n_workers * _CHUNK)

    table128 = _build_pad(vocab, emb_dim)(embedding_weight)

    idx = jnp.maximum(input_variable, 0).reshape(n_workers, chunks, _CHUNK)
    out = _build_gather(total, emb_dim, n_workers, info.num_cores)(idx, table128)
    return out.reshape(batch, hist, emb_dim)
